# Initial kernel scaffold; baseline (speedup 1.0000x reference)
import jax
import jax.numpy as jnp
from jax.experimental import pallas as pl


def kernel(actions, emb_table, W, b):
    raise NotImplementedError("write your pallas kernel here")


# speedup vs baseline: 33.3374x; 33.3374x over previous
import functools

import jax
import jax.numpy as jnp
from jax.experimental import pallas as pl

_A = 7
_V = 256
_H = 1024
_B = 16384
_BS = 512


def _fuse_kernel(emb_ref, w_ref, m_ref):
    m_ref[...] = jax.lax.dot_general(
        emb_ref[...], w_ref[...], (((1,), (1,)), ((), ())),
        preferred_element_type=jnp.float32)[None]


def _bag_kernel(act_ref, m_ref, b_ref, out_ref):
    a = jnp.clip(act_ref[...], -1.0, 1.0)
    tok = ((a + 1.0) * (0.5 * (_V - 1))).astype(jnp.int32)
    acc = jnp.broadcast_to(b_ref[...], (_BS, _H))
    iota = jax.lax.broadcasted_iota(jnp.int32, (_BS, _V), 1)
    for d in range(_A):
        oh = (iota == tok[:, d:d + 1]).astype(jnp.float32)
        acc = acc + jax.lax.dot_general(
            oh, m_ref[d], (((1,), (0,)), ((), ())),
            preferred_element_type=jnp.float32)
    out_ref[...] = acc


def kernel(actions, emb_table, W, b):
    m = pl.pallas_call(
        _fuse_kernel,
        grid=(_A,),
        in_specs=[
            pl.BlockSpec((_V, _H), lambda d: (0, 0)),
            pl.BlockSpec((_H, _H), lambda d: (0, d)),
        ],
        out_specs=pl.BlockSpec((1, _V, _H), lambda d: (d, 0, 0)),
        out_shape=jax.ShapeDtypeStruct((_A, _V, _H), jnp.float32),
    )(emb_table, W)

    out = pl.pallas_call(
        _bag_kernel,
        grid=(_B // _BS,),
        in_specs=[
            pl.BlockSpec((_BS, _A), lambda i: (i, 0)),
            pl.BlockSpec((_A, _V, _H), lambda i: (0, 0, 0)),
            pl.BlockSpec((1, _H), lambda i: (0, 0)),
        ],
        out_specs=pl.BlockSpec((_BS, _H), lambda i: (i, 0)),
        out_shape=jax.ShapeDtypeStruct((_B, _H), jnp.float32),
    )(actions, m, b.reshape(1, _H))
    return out
